# baseline (speedup 1.0000x reference)
import functools

import jax
import jax.numpy as jnp
from jax import lax
from jax.experimental import pallas as pl
from jax.experimental.pallas import tpu as pltpu
from jax.experimental.pallas import tpu_sc as plsc

NC = 2
NS = 16
NW = NC * NS
LANES = 16

B_, S_, D_, V_ = 4, 8192, 1024, 64
R_ = B_ * S_
RPW = R_ // NW
C_ = 16
NCHUNK = RPW // C_
KV = D_ // LANES
NBUF = 3
LOOK = 2


def _sc_body(x_hbm, idx_hbm, table_hbm, out_hbm, *scr):
    table_v, idx_v = scr[0], scr[1]
    ibufs = scr[2:2 + NBUF]
    in_sems = scr[2 + NBUF:2 + 2 * NBUF]
    out_sems = scr[2 + 2 * NBUF:2 + 3 * NBUF]
    wid = lax.axis_index("s") * NC + lax.axis_index("c")
    base = wid * RPW

    for j in range(LOOK):
        pltpu.async_copy(x_hbm.at[pl.ds(base + j * C_, C_)], ibufs[j], in_sems[j])
    pltpu.sync_copy(table_hbm, table_v)
    pltpu.sync_copy(idx_hbm.at[pl.ds(base, RPW)], idx_v.at[pl.ds(0, RPW)])

    def chunk_body(g, b, tail):
        row0 = base + g * C_
        ib = ibufs[b]

        pltpu.make_async_copy(x_hbm.at[pl.ds(row0, C_)], ib, in_sems[b]).wait()

        br = (b + LOOK) % NBUF
        ibr = ibufs[br]

        @pl.when(g >= NBUF - LOOK)
        def _():
            pltpu.make_async_copy(
                ibr, out_hbm.at[pl.ds(row0 - (NBUF - LOOK) * C_, C_)],
                out_sems[br]).wait()

        if not tail:
            @pl.when(g + LOOK < NCHUNK)
            def _():
                pltpu.async_copy(
                    x_hbm.at[pl.ds(row0 + LOOK * C_, C_)], ibr, in_sems[br])

        def row_body(r, c):
            t = idx_v[pl.ds(g * C_ + r, LANES)][0]
            G = 8
            for k0 in range(0, KV, G):
                sls = [pl.ds((k0 + j) * LANES, LANES) for j in range(G)]
                tvs = [table_v[t, sls[j]] for j in range(G)]
                for j in range(G):
                    plsc.addupdate(ib.at[r, sls[j]], tvs[j])
            return c
        lax.fori_loop(0, C_, row_body, 0)

        pltpu.async_copy(ib, out_hbm.at[pl.ds(row0, C_)], out_sems[b])

    NFULL = (NCHUNK // NBUF) * NBUF

    def outer(g0, carry):
        for b in range(NBUF):
            chunk_body(g0 * NBUF + b, b, tail=False)
        return carry

    lax.fori_loop(0, NFULL // NBUF, outer, 0)

    for g in range(NFULL, NCHUNK):
        chunk_body(g, g % NBUF, tail=True)

    for g in range(NCHUNK - (NBUF - LOOK), NCHUNK):
        b = g % NBUF
        row0 = base + g * C_
        pltpu.make_async_copy(
            ibufs[b], out_hbm.at[pl.ds(row0, C_)], out_sems[b]).wait()


@functools.partial(jax.jit, static_argnames=())
def _sc_call(x2, idx, table):
    mesh = plsc.VectorSubcoreMesh(
        core_axis_name="c", subcore_axis_name="s",
        num_cores=NC, num_subcores=NS)
    return pl.kernel(
        _sc_body,
        out_type=jax.ShapeDtypeStruct((R_, D_), jnp.float32),
        mesh=mesh,
        scratch_types=(
            [pltpu.VMEM((V_, D_), jnp.float32),
             pltpu.VMEM((RPW + LANES,), jnp.int32)]
            + [pltpu.VMEM((C_, D_), jnp.float32) for _ in range(NBUF)]
            + [pltpu.SemaphoreType.DMA for _ in range(2 * NBUF)]
        ),
    )(x2, idx, table)


def kernel(x, indices, table):
    x2 = x.reshape(R_, D_)
    idx = indices.reshape(R_).astype(jnp.int32)
    out = _sc_call(x2, idx, table)
    return out.reshape(B_, S_, D_)


# speedup vs baseline: 1.1510x; 1.1510x over previous
import functools

import jax
import jax.numpy as jnp
from jax import lax
from jax.experimental import pallas as pl
from jax.experimental.pallas import tpu as pltpu
from jax.experimental.pallas import tpu_sc as plsc

NC = 2
NS = 16
NW = NC * NS
LANES = 16

B_, S_, D_, V_ = 4, 8192, 1024, 64
R_ = B_ * S_
RPW = R_ // NW
C_ = 4
NCHUNK = RPW // C_
KV = D_ // LANES
NBUF = 14
LOOK = 10


def _sc_body(x_hbm, idx_hbm, table_hbm, out_hbm, *scr):
    table_v, idx_v = scr[0], scr[1]
    ibufs = scr[2:2 + NBUF]
    in_sems = scr[2 + NBUF:2 + 2 * NBUF]
    out_sems = scr[2 + 2 * NBUF:2 + 3 * NBUF]
    wid = lax.axis_index("s") * NC + lax.axis_index("c")
    base = wid * RPW

    for j in range(LOOK):
        pltpu.async_copy(x_hbm.at[pl.ds(base + j * C_, C_)], ibufs[j], in_sems[j])
    pltpu.sync_copy(table_hbm, table_v)
    pltpu.sync_copy(idx_hbm.at[pl.ds(base, RPW)], idx_v.at[pl.ds(0, RPW)])

    def chunk_body(g, b, tail):
        row0 = base + g * C_
        ib = ibufs[b]

        pltpu.make_async_copy(x_hbm.at[pl.ds(row0, C_)], ib, in_sems[b]).wait()

        br = (b + LOOK) % NBUF
        ibr = ibufs[br]

        @pl.when(g >= NBUF - LOOK)
        def _():
            pltpu.make_async_copy(
                ibr, out_hbm.at[pl.ds(row0 - (NBUF - LOOK) * C_, C_)],
                out_sems[br]).wait()

        if not tail:
            @pl.when(g + LOOK < NCHUNK)
            def _():
                pltpu.async_copy(
                    x_hbm.at[pl.ds(row0 + LOOK * C_, C_)], ibr, in_sems[br])

        def row_body(r, c):
            t = idx_v[pl.ds(g * C_ + r, LANES)][0]
            G = 8
            for k0 in range(0, KV, G):
                sls = [pl.ds((k0 + j) * LANES, LANES) for j in range(G)]
                tvs = [table_v[t, sls[j]] for j in range(G)]
                for j in range(G):
                    plsc.addupdate(ib.at[r, sls[j]], tvs[j])
            return c
        lax.fori_loop(0, C_, row_body, 0)

        pltpu.async_copy(ib, out_hbm.at[pl.ds(row0, C_)], out_sems[b])

    NFULL = (NCHUNK // NBUF) * NBUF

    def outer(g0, carry):
        for b in range(NBUF):
            chunk_body(g0 * NBUF + b, b, tail=False)
        return carry

    lax.fori_loop(0, NFULL // NBUF, outer, 0)

    for g in range(NFULL, NCHUNK):
        chunk_body(g, g % NBUF, tail=True)

    for g in range(NCHUNK - (NBUF - LOOK), NCHUNK):
        b = g % NBUF
        row0 = base + g * C_
        pltpu.make_async_copy(
            ibufs[b], out_hbm.at[pl.ds(row0, C_)], out_sems[b]).wait()


@functools.partial(jax.jit, static_argnames=())
def _sc_call(x2, idx, table):
    mesh = plsc.VectorSubcoreMesh(
        core_axis_name="c", subcore_axis_name="s",
        num_cores=NC, num_subcores=NS)
    return pl.kernel(
        _sc_body,
        out_type=jax.ShapeDtypeStruct((R_, D_), jnp.float32),
        mesh=mesh,
        scratch_types=(
            [pltpu.VMEM((V_, D_), jnp.float32),
             pltpu.VMEM((RPW + LANES,), jnp.int32)]
            + [pltpu.VMEM((C_, D_), jnp.float32) for _ in range(NBUF)]
            + [pltpu.SemaphoreType.DMA for _ in range(2 * NBUF)]
        ),
    )(x2, idx, table)


def kernel(x, indices, table):
    x2 = x.reshape(R_, D_)
    idx = indices.reshape(R_).astype(jnp.int32)
    out = _sc_call(x2, idx, table)
    return out.reshape(B_, S_, D_)


# speedup vs baseline: 1.3089x; 1.1372x over previous
import functools

import jax
import jax.numpy as jnp
from jax import lax
from jax.experimental import pallas as pl
from jax.experimental.pallas import tpu as pltpu
from jax.experimental.pallas import tpu_sc as plsc

NC = 2
NS = 16
NW = NC * NS
LANES = 16

B_, S_, D_, V_ = 4, 8192, 1024, 64
R_ = B_ * S_
RPW = R_ // NW
C_ = 8
NCHUNK = RPW // C_
KV = D_ // LANES
NBUF = 7
LOOK = 5


def _sc_body(x_hbm, idx_hbm, table_hbm, out_hbm, *scr):
    table_v, idx_v = scr[0], scr[1]
    ibufs = scr[2:2 + NBUF]
    in_sems = scr[2 + NBUF:2 + 2 * NBUF]
    out_sems = scr[2 + 2 * NBUF:2 + 3 * NBUF]
    wid = lax.axis_index("s") * NC + lax.axis_index("c")
    base = wid * RPW

    for j in range(LOOK):
        pltpu.async_copy(x_hbm.at[pl.ds(base + j * C_, C_)], ibufs[j], in_sems[j])
    pltpu.sync_copy(table_hbm, table_v)
    pltpu.sync_copy(idx_hbm.at[pl.ds(base, RPW)], idx_v.at[pl.ds(0, RPW)])

    def chunk_body(g, b, tail):
        row0 = base + g * C_
        ib = ibufs[b]

        pltpu.make_async_copy(x_hbm.at[pl.ds(row0, C_)], ib, in_sems[b]).wait()

        br = (b + LOOK) % NBUF
        ibr = ibufs[br]

        @pl.when(g >= NBUF - LOOK)
        def _():
            pltpu.make_async_copy(
                ibr, out_hbm.at[pl.ds(row0 - (NBUF - LOOK) * C_, C_)],
                out_sems[br]).wait()

        if not tail:
            @pl.when(g + LOOK < NCHUNK)
            def _():
                pltpu.async_copy(
                    x_hbm.at[pl.ds(row0 + LOOK * C_, C_)], ibr, in_sems[br])

        def row_body(r, c):
            t = idx_v[pl.ds(g * C_ + r, LANES)][0]
            G = 8
            for k0 in range(0, KV, G):
                sls = [pl.ds((k0 + j) * LANES, LANES) for j in range(G)]
                tvs = [table_v[t, sls[j]] for j in range(G)]
                for j in range(G):
                    plsc.addupdate(ib.at[r, sls[j]], tvs[j])
            return c
        lax.fori_loop(0, C_, row_body, 0)

        pltpu.async_copy(ib, out_hbm.at[pl.ds(row0, C_)], out_sems[b])

    NFULL = (NCHUNK // NBUF) * NBUF

    def outer(g0, carry):
        for b in range(NBUF):
            chunk_body(g0 * NBUF + b, b, tail=False)
        return carry

    lax.fori_loop(0, NFULL // NBUF, outer, 0)

    for g in range(NFULL, NCHUNK):
        chunk_body(g, g % NBUF, tail=True)

    for g in range(NCHUNK - (NBUF - LOOK), NCHUNK):
        b = g % NBUF
        row0 = base + g * C_
        pltpu.make_async_copy(
            ibufs[b], out_hbm.at[pl.ds(row0, C_)], out_sems[b]).wait()


@functools.partial(jax.jit, static_argnames=())
def _sc_call(x2, idx, table):
    mesh = plsc.VectorSubcoreMesh(
        core_axis_name="c", subcore_axis_name="s",
        num_cores=NC, num_subcores=NS)
    return pl.kernel(
        _sc_body,
        out_type=jax.ShapeDtypeStruct((R_, D_), jnp.float32),
        mesh=mesh,
        scratch_types=(
            [pltpu.VMEM((V_, D_), jnp.float32),
             pltpu.VMEM((RPW + LANES,), jnp.int32)]
            + [pltpu.VMEM((C_, D_), jnp.float32) for _ in range(NBUF)]
            + [pltpu.SemaphoreType.DMA for _ in range(2 * NBUF)]
        ),
    )(x2, idx, table)


def kernel(x, indices, table):
    x2 = x.reshape(R_, D_)
    idx = indices.reshape(R_).astype(jnp.int32)
    out = _sc_call(x2, idx, table)
    return out.reshape(B_, S_, D_)
